# Initial kernel scaffold; baseline (speedup 1.0000x reference)
import jax
import jax.numpy as jnp
from jax.experimental import pallas as pl


def kernel(z_t0, t, sqrt_alphas_cumprod, sqrt_one_minus_alphas_cumprod, noise):
    raise NotImplementedError("write your pallas kernel here")


# speedup vs baseline: 1.1214x; 1.1214x over previous
import functools

import jax
import jax.numpy as jnp
from jax import lax
from jax.experimental import pallas as pl
from jax.experimental.pallas import tpu as pltpu
from jax.experimental.pallas import tpu_sc as plsc

B = 16384
D = 512
NC = 2
NS = 16
NW = NC * NS
ROWS_PER_W = B // NW
CHUNK = 32
NCHUNK = ROWS_PER_W // CHUNK
LANES = 16
VREGS_PER_CHUNK = CHUNK * D // LANES


def _sc_body(z_hbm, t_hbm, ac_hbm, omac_hbm, noise_hbm,
             zt_hbm, mu_hbm, sig_hbm,
             idx_v, z_v, noise_v, ga_v, gb_v, sem):
    wid = lax.axis_index("s") * NC + lax.axis_index("c")
    row0 = wid * ROWS_PER_W

    def chunk_body(i, carry):
        base = row0 + i * CHUNK
        pltpu.sync_copy(t_hbm.at[pl.ds(base, CHUNK)], idx_v)
        pltpu.async_copy(ac_hbm.at[idx_v], ga_v, sem).wait()
        pltpu.async_copy(omac_hbm.at[idx_v], gb_v, sem).wait()
        pltpu.sync_copy(z_hbm.at[pl.ds(base, CHUNK)], z_v)
        pltpu.sync_copy(noise_hbm.at[pl.ds(base, CHUNK)], noise_v)

        def vec_body(k, c):
            r = k >> 5
            col = (k & 31) * LANES
            a = ga_v[r, pl.ds(col, LANES)]
            b = gb_v[r, pl.ds(col, LANES)]
            z = z_v[r, pl.ds(col, LANES)]
            n = noise_v[r, pl.ds(col, LANES)]
            mu = a * z
            ga_v[r, pl.ds(col, LANES)] = mu
            z_v[r, pl.ds(col, LANES)] = mu + n * b
            return c

        lax.fori_loop(0, VREGS_PER_CHUNK, vec_body, 0, unroll=4)

        pltpu.sync_copy(z_v, zt_hbm.at[pl.ds(base, CHUNK)])
        pltpu.sync_copy(ga_v, mu_hbm.at[pl.ds(base, CHUNK)])
        pltpu.sync_copy(gb_v, sig_hbm.at[pl.ds(base, CHUNK)])
        return carry

    lax.fori_loop(0, NCHUNK, chunk_body, 0)


def kernel(z_t0, t, sqrt_alphas_cumprod, sqrt_one_minus_alphas_cumprod, noise):
    mesh = plsc.VectorSubcoreMesh(core_axis_name="c", subcore_axis_name="s")
    out_sds = jax.ShapeDtypeStruct((B, D), jnp.float32)
    fn = functools.partial(
        pl.kernel,
        out_type=(out_sds, out_sds, out_sds),
        mesh=mesh,
        scratch_types=[
            pltpu.VMEM((CHUNK,), jnp.int32),
            pltpu.VMEM((CHUNK, D), jnp.float32),
            pltpu.VMEM((CHUNK, D), jnp.float32),
            pltpu.VMEM((CHUNK, D), jnp.float32),
            pltpu.VMEM((CHUNK, D), jnp.float32),
            pltpu.SemaphoreType.DMA,
        ],
    )(_sc_body)
    z_t, mu, sigma = fn(z_t0, t, sqrt_alphas_cumprod,
                        sqrt_one_minus_alphas_cumprod, noise)
    return (z_t, mu, sigma)


# speedup vs baseline: 1.8001x; 1.6052x over previous
import functools

import jax
import jax.numpy as jnp
from jax import lax
from jax.experimental import pallas as pl
from jax.experimental.pallas import tpu as pltpu
from jax.experimental.pallas import tpu_sc as plsc

B = 16384
D = 512
NC = 2
NS = 16
NW = NC * NS
ROWS_PER_W = B // NW
CHUNK = 16
NCHUNK = ROWS_PER_W // CHUNK
NBUF = 3
LANES = 16
VREGS_PER_CHUNK = CHUNK * D // LANES


def _sc_body(z_hbm, t_hbm, ac_hbm, omac_hbm, noise_hbm,
             zt_hbm, mu_hbm, sig_hbm,
             idx_all, z_v, noise_v, ga_v, gb_v, sem_in, sem_out):
    wid = lax.axis_index("s") * NC + lax.axis_index("c")
    row0 = wid * ROWS_PER_W

    pltpu.sync_copy(t_hbm.at[pl.ds(row0, ROWS_PER_W)], idx_all)

    def start_in(i):
        b = i % NBUF
        idx = idx_all.at[pl.ds(i * CHUNK, CHUNK)]
        rows = pl.ds(row0 + i * CHUNK, CHUNK)
        return [
            pltpu.async_copy(ac_hbm.at[idx], ga_v.at[b], sem_in.at[b]),
            pltpu.async_copy(omac_hbm.at[idx], gb_v.at[b], sem_in.at[b]),
            pltpu.async_copy(z_hbm.at[rows], z_v.at[b], sem_in.at[b]),
            pltpu.async_copy(noise_hbm.at[rows], noise_v.at[b], sem_in.at[b]),
        ]

    def compute(b):
        def vec_body(k, c):
            r = k >> 5
            col = (k & 31) * LANES
            a = ga_v[b, r, pl.ds(col, LANES)]
            gb = gb_v[b, r, pl.ds(col, LANES)]
            z = z_v[b, r, pl.ds(col, LANES)]
            n = noise_v[b, r, pl.ds(col, LANES)]
            mu = a * z
            ga_v[b, r, pl.ds(col, LANES)] = mu
            z_v[b, r, pl.ds(col, LANES)] = mu + n * gb
            return c

        lax.fori_loop(0, VREGS_PER_CHUNK, vec_body, 0, unroll=8)

    in_descs = {}
    out_descs = {}
    in_descs[0] = start_in(0)
    for i in range(NCHUNK):
        b = i % NBUF
        if i + 1 < NCHUNK:
            if i - 2 >= 0:
                for d in out_descs[i - 2]:
                    d.wait()
            in_descs[i + 1] = start_in(i + 1)
        for d in in_descs[i]:
            d.wait()
        rows = pl.ds(row0 + i * CHUNK, CHUNK)
        sig_d = pltpu.async_copy(gb_v.at[b], sig_hbm.at[rows], sem_out.at[b])
        compute(b)
        out_descs[i] = [
            sig_d,
            pltpu.async_copy(z_v.at[b], zt_hbm.at[rows], sem_out.at[b]),
            pltpu.async_copy(ga_v.at[b], mu_hbm.at[rows], sem_out.at[b]),
        ]
    for i in range(max(0, NCHUNK - 3), NCHUNK):
        for d in out_descs[i]:
            d.wait()


def kernel(z_t0, t, sqrt_alphas_cumprod, sqrt_one_minus_alphas_cumprod, noise):
    mesh = plsc.VectorSubcoreMesh(core_axis_name="c", subcore_axis_name="s")
    out_sds = jax.ShapeDtypeStruct((B, D), jnp.float32)
    fn = functools.partial(
        pl.kernel,
        out_type=(out_sds, out_sds, out_sds),
        mesh=mesh,
        scratch_types=[
            pltpu.VMEM((ROWS_PER_W,), jnp.int32),
            pltpu.VMEM((NBUF, CHUNK, D), jnp.float32),
            pltpu.VMEM((NBUF, CHUNK, D), jnp.float32),
            pltpu.VMEM((NBUF, CHUNK, D), jnp.float32),
            pltpu.VMEM((NBUF, CHUNK, D), jnp.float32),
            pltpu.SemaphoreType.DMA((NBUF,)),
            pltpu.SemaphoreType.DMA((NBUF,)),
        ],
    )(_sc_body)
    z_t, mu, sigma = fn(z_t0, t, sqrt_alphas_cumprod,
                        sqrt_one_minus_alphas_cumprod, noise)
    return (z_t, mu, sigma)
